# baseline (speedup 1.0000x reference)
import functools

import jax
import jax.numpy as jnp
from jax import lax
from jax.experimental import pallas as pl
from jax.experimental.pallas import tpu as pltpu
from jax.experimental.pallas import tpu_sc as plsc

_NC = 2
_NS = 16
_NW = _NC * _NS

_CHUNK = 128
_PAD = 128
_TCOLS = 2048


def _transpose_body(dim, in_ref, out_ref):
    x = in_ref[...]
    y = x.T
    out_ref[:, :dim] = y
    out_ref[:, dim:] = jnp.zeros_like(out_ref[:, dim:])


def _tc_transpose(table_t, vocab_pad):
    dim, vocab = table_t.shape
    grid = (vocab_pad // _TCOLS,)
    return pl.pallas_call(
        functools.partial(_transpose_body, dim),
        grid=grid,
        in_specs=[pl.BlockSpec((dim, _TCOLS), lambda i: (0, i))],
        out_specs=pl.BlockSpec((_TCOLS, _PAD), lambda i: (i, 0)),
        out_shape=jax.ShapeDtypeStruct((vocab_pad, _PAD), jnp.float32),
    )(table_t)


def _gather_body(n_tok, n_chunks, sen_hbm, table_hbm, out_hbm,
                 idx_v, rows0, rows1, sem0, sem1):
    wid = lax.axis_index("s") * _NC + lax.axis_index("c")
    base = wid * n_tok

    pltpu.sync_copy(sen_hbm.at[pl.ds(base, n_tok)], idx_v)

    pltpu.async_copy(table_hbm.at[idx_v.at[pl.ds(0, _CHUNK)]], rows0, sem0)
    pltpu.async_copy(table_hbm.at[idx_v.at[pl.ds(_CHUNK, _CHUNK)]], rows1, sem1)

    def body(i, carry):
        for b, (buf, sem) in enumerate(((rows0, sem0), (rows1, sem1))):
            g = 2 * i + b
            pltpu.make_async_copy(
                table_hbm.at[idx_v.at[pl.ds(g * _CHUNK, _CHUNK)]], buf, sem
            ).wait()
            pltpu.sync_copy(buf, out_hbm.at[pl.ds(base + g * _CHUNK, _CHUNK)])

            @pl.when(g + 2 < n_chunks)
            def _():
                pltpu.async_copy(
                    table_hbm.at[idx_v.at[pl.ds((g + 2) * _CHUNK, _CHUNK)]],
                    buf, sem)
        return carry

    lax.fori_loop(0, n_chunks // 2, body, 0)


@functools.partial(jax.jit, static_argnames=("n_tok_total",))
def _sc_gather(sen_flat, table_pad, n_tok_total):
    n_tok = n_tok_total // _NW
    n_chunks = n_tok // _CHUNK
    mesh = plsc.VectorSubcoreMesh(core_axis_name="c", subcore_axis_name="s")
    return pl.kernel(
        functools.partial(_gather_body, n_tok, n_chunks),
        out_type=jax.ShapeDtypeStruct((n_tok_total, _PAD), jnp.float32),
        mesh=mesh,
        scratch_types=[
            pltpu.VMEM((n_tok,), jnp.int32),
            pltpu.VMEM((_CHUNK, _PAD), jnp.float32),
            pltpu.VMEM((_CHUNK, _PAD), jnp.float32),
            pltpu.SemaphoreType.DMA,
            pltpu.SemaphoreType.DMA,
        ],
    )(sen_flat, table_pad)


_TB = 3200


def _ln_body(dim, rows_ref, pos_ref, gamma_ref, beta_ref, out_ref):
    x = rows_ref[...][:, :dim] + pos_ref[...]
    mean = jnp.mean(x, axis=-1, keepdims=True)
    c = x - mean
    var = jnp.mean(c * c, axis=-1, keepdims=True)
    y = (c * lax.rsqrt(var + 1e-6)) * gamma_ref[...] + beta_ref[...]
    out_ref[...] = y.T


def _tc_layernorm(gathered, pos_tiled, gamma, beta):
    n_tok, dp = gathered.shape
    dim = pos_tiled.shape[-1]
    grid = (n_tok // _TB,)
    return pl.pallas_call(
        functools.partial(_ln_body, dim),
        grid=grid,
        in_specs=[
            pl.BlockSpec((_TB, dp), lambda i: (i, 0)),
            pl.BlockSpec((_TB, dim), lambda i: (0, 0)),
            pl.BlockSpec((1, dim), lambda i: (0, 0)),
            pl.BlockSpec((1, dim), lambda i: (0, 0)),
        ],
        out_specs=pl.BlockSpec((dim, _TB), lambda i: (0, i)),
        out_shape=jax.ShapeDtypeStruct((dim, n_tok), jnp.float32),
    )(gathered, pos_tiled, gamma, beta)


def kernel(sen, table, pos_emb, gamma, beta):
    b, l = sen.shape
    vocab, dim = table.shape
    vocab_pad = ((vocab + _TCOLS - 1) // _TCOLS) * _TCOLS
    sen_flat = sen.reshape(-1).astype(jnp.int32)
    table_pad = _tc_transpose(table.T, vocab_pad)
    gathered = _sc_gather(sen_flat, table_pad, b * l)
    pos_tiled = jnp.tile(pos_emb[:l], (_TB // l, 1))
    out_t = _tc_layernorm(
        gathered,
        pos_tiled,
        gamma.reshape(1, dim),
        beta.reshape(1, dim),
    )
    return out_t.T.reshape(b, l, dim)


# speedup vs baseline: 1.8477x; 1.8477x over previous
import functools

import jax
import jax.numpy as jnp
from jax import lax
from jax.experimental import pallas as pl
from jax.experimental.pallas import tpu as pltpu
from jax.experimental.pallas import tpu_sc as plsc

_NC = 2
_NS = 16
_NW = _NC * _NS

_CHUNK = 128
_PAD = 128
_TCOLS = 2048


def _transpose_body(dim, in_ref, out_ref):
    x = in_ref[...]
    y = x.T
    out_ref[:, :dim] = y
    out_ref[:, dim:] = jnp.zeros_like(out_ref[:, dim:])


def _tc_transpose(table_t, vocab_pad):
    dim, vocab = table_t.shape
    grid = (vocab_pad // _TCOLS,)
    return pl.pallas_call(
        functools.partial(_transpose_body, dim),
        grid=grid,
        in_specs=[pl.BlockSpec((dim, _TCOLS), lambda i: (0, i))],
        out_specs=pl.BlockSpec((_TCOLS, _PAD), lambda i: (i, 0)),
        out_shape=jax.ShapeDtypeStruct((vocab_pad, _PAD), jnp.float32),
    )(table_t)


def _gather_body(n_tok, n_chunks, sen_hbm, table_hbm, out_hbm,
                 idx_v, rows0, rows1, sem0, sem1):
    wid = lax.axis_index("s") * _NC + lax.axis_index("c")
    base = wid * n_tok

    pltpu.sync_copy(sen_hbm.at[pl.ds(base, n_tok)], idx_v)

    pltpu.async_copy(table_hbm.at[idx_v.at[pl.ds(0, _CHUNK)]], rows0, sem0)
    pltpu.async_copy(table_hbm.at[idx_v.at[pl.ds(_CHUNK, _CHUNK)]], rows1, sem1)

    def body(i, carry):
        for b, (buf, sem) in enumerate(((rows0, sem0), (rows1, sem1))):
            g = 2 * i + b
            pltpu.make_async_copy(
                table_hbm.at[idx_v.at[pl.ds(g * _CHUNK, _CHUNK)]], buf, sem
            ).wait()
            pltpu.sync_copy(buf, out_hbm.at[pl.ds(base + g * _CHUNK, _CHUNK)])

            @pl.when(g + 2 < n_chunks)
            def _():
                pltpu.async_copy(
                    table_hbm.at[idx_v.at[pl.ds((g + 2) * _CHUNK, _CHUNK)]],
                    buf, sem)
        return carry

    lax.fori_loop(0, n_chunks // 2, body, 0)


@functools.partial(jax.jit, static_argnames=("n_tok_total",))
def _sc_gather(sen_flat, table_pad, n_tok_total):
    n_tok = n_tok_total // _NW
    n_chunks = n_tok // _CHUNK
    mesh = plsc.VectorSubcoreMesh(core_axis_name="c", subcore_axis_name="s")
    return pl.kernel(
        functools.partial(_gather_body, n_tok, n_chunks),
        out_type=jax.ShapeDtypeStruct((n_tok_total, _PAD), jnp.float32),
        mesh=mesh,
        scratch_types=[
            pltpu.VMEM((n_tok,), jnp.int32),
            pltpu.VMEM((_CHUNK, _PAD), jnp.float32),
            pltpu.VMEM((_CHUNK, _PAD), jnp.float32),
            pltpu.SemaphoreType.DMA,
            pltpu.SemaphoreType.DMA,
        ],
    )(sen_flat, table_pad)


_LBLK = 8


def _ln_body(dim, b, rows_ref, pos_ref, gamma_ref, beta_ref, out_ref):
    x = rows_ref[...][:, :dim] + pos_ref[...].reshape(_LBLK, 1, dim).repeat(
        b, axis=1).reshape(_LBLK * b, dim)
    mean = jnp.mean(x, axis=-1, keepdims=True)
    c = x - mean
    var = jnp.mean(c * c, axis=-1, keepdims=True)
    y = (c * lax.rsqrt(var + 1e-6)) * gamma_ref[...] + beta_ref[...]
    for j in range(_LBLK):
        out_ref[:, j, :] = y[j * b:(j + 1) * b, :].T


def _tc_layernorm(gathered, pos, gamma, beta, b, l):
    n_tok, dp = gathered.shape
    dim = pos.shape[-1]
    grid = (l // _LBLK,)
    return pl.pallas_call(
        functools.partial(_ln_body, dim, b),
        grid=grid,
        in_specs=[
            pl.BlockSpec((_LBLK * b, dp), lambda i: (i, 0)),
            pl.BlockSpec((_LBLK, dim), lambda i: (i, 0)),
            pl.BlockSpec((1, dim), lambda i: (0, 0)),
            pl.BlockSpec((1, dim), lambda i: (0, 0)),
        ],
        out_specs=pl.BlockSpec((dim, _LBLK, b), lambda i: (0, i, 0)),
        out_shape=jax.ShapeDtypeStruct((dim, l, b), jnp.float32),
    )(gathered, pos, gamma, beta)


def kernel(sen, table, pos_emb, gamma, beta):
    b, l = sen.shape
    vocab, dim = table.shape
    vocab_pad = ((vocab + _TCOLS - 1) // _TCOLS) * _TCOLS
    sen_flat = sen.T.reshape(-1).astype(jnp.int32)
    table_pad = _tc_transpose(table.T, vocab_pad)
    gathered = _sc_gather(sen_flat, table_pad, b * l)
    out_t = _tc_layernorm(
        gathered,
        pos_emb[:l],
        gamma.reshape(1, dim),
        beta.reshape(1, dim),
        b, l,
    )
    return out_t.transpose(2, 1, 0)


# speedup vs baseline: 2.0954x; 1.1341x over previous
import functools

import jax
import jax.numpy as jnp
from jax import lax
from jax.experimental import pallas as pl
from jax.experimental.pallas import tpu as pltpu
from jax.experimental.pallas import tpu_sc as plsc

_NC = 2
_NS = 16
_NW = _NC * _NS

_CHUNK = 128
_PAD = 128
_TCOLS = 4096


def _transpose_body(dim, in_ref, out_ref):
    x = in_ref[...]
    y = x.T
    out_ref[:, :dim] = y
    out_ref[:, dim:] = jnp.zeros_like(out_ref[:, dim:])


def _tc_transpose(table_t, vocab_pad):
    dim, vocab = table_t.shape
    grid = (vocab_pad // _TCOLS,)
    return pl.pallas_call(
        functools.partial(_transpose_body, dim),
        grid=grid,
        in_specs=[pl.BlockSpec((dim, _TCOLS), lambda i: (0, i))],
        out_specs=pl.BlockSpec((_TCOLS, _PAD), lambda i: (i, 0)),
        out_shape=jax.ShapeDtypeStruct((vocab_pad, _PAD), jnp.float32),
    )(table_t)


def _gather_body(n_tok, n_chunks, sen_hbm, table_hbm, out_hbm,
                 idx_v, rows0, rows1, sem0, sem1):
    wid = lax.axis_index("s") * _NC + lax.axis_index("c")
    base = wid * n_tok

    pltpu.sync_copy(sen_hbm.at[pl.ds(base, n_tok)], idx_v)

    pltpu.async_copy(table_hbm.at[idx_v.at[pl.ds(0, _CHUNK)]], rows0, sem0)
    pltpu.async_copy(table_hbm.at[idx_v.at[pl.ds(_CHUNK, _CHUNK)]], rows1, sem1)

    def body(i, carry):
        for b, (buf, sem) in enumerate(((rows0, sem0), (rows1, sem1))):
            g = 2 * i + b
            pltpu.make_async_copy(
                table_hbm.at[idx_v.at[pl.ds(g * _CHUNK, _CHUNK)]], buf, sem
            ).wait()
            pltpu.sync_copy(buf, out_hbm.at[pl.ds(base + g * _CHUNK, _CHUNK)])

            @pl.when(g + 2 < n_chunks)
            def _():
                pltpu.async_copy(
                    table_hbm.at[idx_v.at[pl.ds((g + 2) * _CHUNK, _CHUNK)]],
                    buf, sem)
        return carry

    lax.fori_loop(0, n_chunks // 2, body, 0)


@functools.partial(jax.jit, static_argnames=("n_tok_total",))
def _sc_gather(sen_flat, table_pad, n_tok_total):
    n_tok = n_tok_total // _NW
    n_chunks = n_tok // _CHUNK
    mesh = plsc.VectorSubcoreMesh(core_axis_name="c", subcore_axis_name="s")
    return pl.kernel(
        functools.partial(_gather_body, n_tok, n_chunks),
        out_type=jax.ShapeDtypeStruct((n_tok_total, _PAD), jnp.float32),
        mesh=mesh,
        scratch_types=[
            pltpu.VMEM((n_tok,), jnp.int32),
            pltpu.VMEM((_CHUNK, _PAD), jnp.float32),
            pltpu.VMEM((_CHUNK, _PAD), jnp.float32),
            pltpu.SemaphoreType.DMA,
            pltpu.SemaphoreType.DMA,
        ],
    )(sen_flat, table_pad)


_LBLK = 8


def _ln_body(dim, b, rows_ref, pos_ref, gamma_ref, beta_ref, out_ref):
    x = rows_ref[...][:, :dim] + pos_ref[...].reshape(_LBLK, 1, dim).repeat(
        b, axis=1).reshape(_LBLK * b, dim)
    mean = jnp.mean(x, axis=-1, keepdims=True)
    c = x - mean
    var = jnp.mean(c * c, axis=-1, keepdims=True)
    y = (c * lax.rsqrt(var + 1e-6)) * gamma_ref[...] + beta_ref[...]
    for j in range(_LBLK):
        out_ref[:, j, :] = y[j * b:(j + 1) * b, :].T


def _tc_layernorm(gathered, pos, gamma, beta, b, l):
    n_tok, dp = gathered.shape
    dim = pos.shape[-1]
    grid = (l // _LBLK,)
    return pl.pallas_call(
        functools.partial(_ln_body, dim, b),
        grid=grid,
        in_specs=[
            pl.BlockSpec((_LBLK * b, dp), lambda i: (i, 0)),
            pl.BlockSpec((_LBLK, dim), lambda i: (i, 0)),
            pl.BlockSpec((1, dim), lambda i: (0, 0)),
            pl.BlockSpec((1, dim), lambda i: (0, 0)),
        ],
        out_specs=pl.BlockSpec((dim, _LBLK, b), lambda i: (0, i, 0)),
        out_shape=jax.ShapeDtypeStruct((dim, l, b), jnp.float32),
    )(gathered, pos, gamma, beta)


def kernel(sen, table, pos_emb, gamma, beta):
    b, l = sen.shape
    vocab, dim = table.shape
    vocab_pad = ((vocab + _TCOLS - 1) // _TCOLS) * _TCOLS
    sen_flat = sen.T.reshape(-1).astype(jnp.int32)
    table_pad = _tc_transpose(table.T, vocab_pad)
    gathered = _sc_gather(sen_flat, table_pad, b * l)
    out_t = _tc_layernorm(
        gathered,
        pos_emb[:l],
        gamma.reshape(1, dim),
        beta.reshape(1, dim),
        b, l,
    )
    return out_t.transpose(2, 1, 0)


# speedup vs baseline: 2.2307x; 1.0646x over previous
import functools

import jax
import jax.numpy as jnp
from jax import lax
from jax.experimental import pallas as pl
from jax.experimental.pallas import tpu as pltpu
from jax.experimental.pallas import tpu_sc as plsc

_NC = 2
_NS = 16
_NW = _NC * _NS

_CHUNK = 128
_PAD = 128
_TCOLS = 8192


def _transpose_body(dim, in_ref, out_ref):
    x = in_ref[...]
    y = x.T
    out_ref[:, :dim] = y
    out_ref[:, dim:] = jnp.zeros_like(out_ref[:, dim:])


def _tc_transpose(table_t, vocab_pad):
    dim, vocab = table_t.shape
    grid = (vocab_pad // _TCOLS,)
    return pl.pallas_call(
        functools.partial(_transpose_body, dim),
        grid=grid,
        in_specs=[pl.BlockSpec((dim, _TCOLS), lambda i: (0, i))],
        out_specs=pl.BlockSpec((_TCOLS, _PAD), lambda i: (i, 0)),
        out_shape=jax.ShapeDtypeStruct((vocab_pad, _PAD), jnp.float32),
    )(table_t)


def _gather_body(n_tok, n_chunks, sen_hbm, table_hbm, out_hbm,
                 idx_v, rows0, rows1, sem0, sem1):
    wid = lax.axis_index("s") * _NC + lax.axis_index("c")
    base = wid * n_tok

    pltpu.sync_copy(sen_hbm.at[pl.ds(base, n_tok)], idx_v)

    pltpu.async_copy(table_hbm.at[idx_v.at[pl.ds(0, _CHUNK)]], rows0, sem0)
    pltpu.async_copy(table_hbm.at[idx_v.at[pl.ds(_CHUNK, _CHUNK)]], rows1, sem1)

    def body(i, carry):
        for b, (buf, sem) in enumerate(((rows0, sem0), (rows1, sem1))):
            g = 2 * i + b
            pltpu.make_async_copy(
                table_hbm.at[idx_v.at[pl.ds(g * _CHUNK, _CHUNK)]], buf, sem
            ).wait()
            pltpu.sync_copy(buf, out_hbm.at[pl.ds(base + g * _CHUNK, _CHUNK)])

            @pl.when(g + 2 < n_chunks)
            def _():
                pltpu.async_copy(
                    table_hbm.at[idx_v.at[pl.ds((g + 2) * _CHUNK, _CHUNK)]],
                    buf, sem)
        return carry

    lax.fori_loop(0, n_chunks // 2, body, 0)


@functools.partial(jax.jit, static_argnames=("n_tok_total",))
def _sc_gather(sen_flat, table_pad, n_tok_total):
    n_tok = n_tok_total // _NW
    n_chunks = n_tok // _CHUNK
    mesh = plsc.VectorSubcoreMesh(core_axis_name="c", subcore_axis_name="s")
    return pl.kernel(
        functools.partial(_gather_body, n_tok, n_chunks),
        out_type=jax.ShapeDtypeStruct((n_tok_total, _PAD), jnp.float32),
        mesh=mesh,
        scratch_types=[
            pltpu.VMEM((n_tok,), jnp.int32),
            pltpu.VMEM((_CHUNK, _PAD), jnp.float32),
            pltpu.VMEM((_CHUNK, _PAD), jnp.float32),
            pltpu.SemaphoreType.DMA,
            pltpu.SemaphoreType.DMA,
        ],
    )(sen_flat, table_pad)


_LBLK = 8


def _ln_body(dim, b, rows_ref, pos_ref, gamma_ref, beta_ref, out_ref):
    x = rows_ref[...][:, :dim] + pos_ref[...].reshape(_LBLK, 1, dim).repeat(
        b, axis=1).reshape(_LBLK * b, dim)
    mean = jnp.mean(x, axis=-1, keepdims=True)
    c = x - mean
    var = jnp.mean(c * c, axis=-1, keepdims=True)
    y = (c * lax.rsqrt(var + 1e-6)) * gamma_ref[...] + beta_ref[...]
    for j in range(_LBLK):
        out_ref[:, j, :] = y[j * b:(j + 1) * b, :].T


def _tc_layernorm(gathered, pos, gamma, beta, b, l):
    n_tok, dp = gathered.shape
    dim = pos.shape[-1]
    grid = (l // _LBLK,)
    return pl.pallas_call(
        functools.partial(_ln_body, dim, b),
        grid=grid,
        in_specs=[
            pl.BlockSpec((_LBLK * b, dp), lambda i: (i, 0)),
            pl.BlockSpec((_LBLK, dim), lambda i: (i, 0)),
            pl.BlockSpec((1, dim), lambda i: (0, 0)),
            pl.BlockSpec((1, dim), lambda i: (0, 0)),
        ],
        out_specs=pl.BlockSpec((dim, _LBLK, b), lambda i: (0, i, 0)),
        out_shape=jax.ShapeDtypeStruct((dim, l, b), jnp.float32),
    )(gathered, pos, gamma, beta)


def kernel(sen, table, pos_emb, gamma, beta):
    b, l = sen.shape
    vocab, dim = table.shape
    vocab_pad = ((vocab + _TCOLS - 1) // _TCOLS) * _TCOLS
    sen_flat = sen.T.reshape(-1).astype(jnp.int32)
    table_pad = _tc_transpose(table.T, vocab_pad)
    gathered = _sc_gather(sen_flat, table_pad, b * l)
    out_t = _tc_layernorm(
        gathered,
        pos_emb[:l],
        gamma.reshape(1, dim),
        beta.reshape(1, dim),
        b, l,
    )
    return out_t.transpose(2, 1, 0)
